# baseline (speedup 1.0000x reference)
import jax
import jax.numpy as jnp
from jax.experimental import pallas as pl
from jax.experimental.pallas import tpu as pltpu

B = 4096
D_IN = 1024
D_HID = 2048
D_EMB = 128

BM = 512


def _body(xq_ref, xd_ref, wq1_hbm, bq1_ref, wq2_hbm, bq2_ref,
          wd1_hbm, bd1_ref, wd2_hbm, bd2_ref, oq_ref, od_ref,
          wq1_v, wq2_v, wd1_v, wd2_v, sq1, sq2, sd1, sd2):
    i = pl.program_id(0)

    @pl.when(i == 0)
    def _start_weight_dma():
        pltpu.make_async_copy(wq1_hbm, wq1_v, sq1).start()
        pltpu.make_async_copy(wq2_hbm, wq2_v, sq2).start()
        pltpu.make_async_copy(wd1_hbm, wd1_v, sd1).start()
        pltpu.make_async_copy(wd2_hbm, wd2_v, sd2).start()

    @pl.when(i == 0)
    def _wait_q_weights():
        pltpu.make_async_copy(wq1_hbm, wq1_v, sq1).wait()
        pltpu.make_async_copy(wq2_hbm, wq2_v, sq2).wait()

    hq = jnp.maximum(
        jnp.dot(xq_ref[:], wq1_v[:], preferred_element_type=jnp.float32)
        + bq1_ref[:], 0.0)
    oq_ref[:] = (jnp.dot(hq, wq2_v[:], preferred_element_type=jnp.float32)
                 + bq2_ref[:])

    @pl.when(i == 0)
    def _wait_d_weights():
        pltpu.make_async_copy(wd1_hbm, wd1_v, sd1).wait()
        pltpu.make_async_copy(wd2_hbm, wd2_v, sd2).wait()

    hd = jnp.maximum(
        jnp.dot(xd_ref[:], wd1_v[:], preferred_element_type=jnp.float32)
        + bd1_ref[:], 0.0)
    od_ref[:] = (jnp.dot(hd, wd2_v[:], preferred_element_type=jnp.float32)
                 + bd2_ref[:])


def kernel(query, document, Wq1, bq1, Wq2, bq2, Wd1, bd1, Wd2, bd2):
    bq1_2d = bq1.reshape(1, D_HID)
    bq2_2d = bq2.reshape(1, D_EMB)
    bd1_2d = bd1.reshape(1, D_HID)
    bd2_2d = bd2.reshape(1, D_EMB)

    x_spec = pl.BlockSpec((BM, D_IN), lambda i: (i, 0))
    w_spec = pl.BlockSpec(memory_space=pl.ANY)
    b1_spec = pl.BlockSpec((1, D_HID), lambda i: (0, 0))
    b2_spec = pl.BlockSpec((1, D_EMB), lambda i: (0, 0))
    o_spec = pl.BlockSpec((BM, D_EMB), lambda i: (i, 0))

    oq, od = pl.pallas_call(
        _body,
        grid=(B // BM,),
        in_specs=[x_spec, x_spec,
                  w_spec, b1_spec, w_spec, b2_spec,
                  w_spec, b1_spec, w_spec, b2_spec],
        out_specs=[o_spec, o_spec],
        out_shape=[jax.ShapeDtypeStruct((B, D_EMB), jnp.float32),
                   jax.ShapeDtypeStruct((B, D_EMB), jnp.float32)],
        scratch_shapes=[
            pltpu.VMEM((D_IN, D_HID), jnp.float32),
            pltpu.VMEM((D_HID, D_EMB), jnp.float32),
            pltpu.VMEM((D_IN, D_HID), jnp.float32),
            pltpu.VMEM((D_HID, D_EMB), jnp.float32),
            pltpu.SemaphoreType.DMA,
            pltpu.SemaphoreType.DMA,
            pltpu.SemaphoreType.DMA,
            pltpu.SemaphoreType.DMA,
        ],
        compiler_params=pltpu.CompilerParams(
            dimension_semantics=("arbitrary",),
        ),
    )(query, document, Wq1, bq1_2d, Wq2, bq2_2d, Wd1, bd1_2d, Wd2, bd2_2d)
    return (oq, od)
